# baseline (speedup 1.0000x reference)
import functools

import jax
import jax.numpy as jnp
from jax import lax
from jax.experimental import pallas as pl
from jax.experimental.pallas import tpu as pltpu
from jax.experimental.pallas import tpu_sc as plsc

N_NODES = 10000
N_EDGES = 320000
D_FEAT = 128
LANES = 16

NC = 2
NS = 16
NW = NC * NS

NPAD = 10240
BM = 512
BN = 2560
EDGES_PER_TILE = N_EDGES // NW


def _mm_body(m_ref, mt_ref, g_ref):
    g_ref[...] = lax.dot_general(
        m_ref[...], mt_ref[...], (((1,), (1,)), ((), ())),
        preferred_element_type=jnp.float32)


def _gram(mp):
    return pl.pallas_call(
        _mm_body,
        grid=(NPAD // BM, NPAD // BN),
        in_specs=[pl.BlockSpec((BM, D_FEAT), lambda i, j: (i, 0)),
                  pl.BlockSpec((BN, D_FEAT), lambda i, j: (j, 0))],
        out_specs=pl.BlockSpec((BM, BN), lambda i, j: (i, j)),
        out_shape=jax.ShapeDtypeStruct((NPAD, NPAD), jnp.float32),
    )(mp, mp)


def _sc_body(src_hbm, dst_hbm, gflat_hbm, out_hbm, sidx, didx, widx, sem):
    cid = lax.axis_index("c")
    sid = lax.axis_index("s")
    wid = sid * NC + cid
    base = wid * EDGES_PER_TILE

    pltpu.sync_copy(src_hbm.at[pl.ds(base, EDGES_PER_TILE)], sidx)
    pltpu.sync_copy(dst_hbm.at[pl.ds(base, EDGES_PER_TILE)], didx)

    def flat_step(i, _):
        s = pl.ds(i * LANES, LANES)
        widx[s] = sidx[s] * NPAD + didx[s]
        return 0

    lax.fori_loop(0, EDGES_PER_TILE // LANES, flat_step, 0)

    pltpu.async_copy(gflat_hbm.at[widx], sidx, sem).wait()
    pltpu.sync_copy(sidx, out_hbm.at[pl.ds(base, EDGES_PER_TILE)])


def _sc_gather(src, dst, gflat):
    mesh = plsc.VectorSubcoreMesh(core_axis_name="c", subcore_axis_name="s")
    return pl.kernel(
        _sc_body,
        out_type=jax.ShapeDtypeStruct((N_EDGES,), jnp.int32),
        mesh=mesh,
        compiler_params=pltpu.CompilerParams(needs_layout_passes=False,
                                             use_tc_tiling_on_sc=False),
        scratch_types=[
            pltpu.VMEM((EDGES_PER_TILE,), jnp.int32),
            pltpu.VMEM((EDGES_PER_TILE,), jnp.int32),
            pltpu.VMEM((EDGES_PER_TILE,), jnp.int32),
            pltpu.SemaphoreType.DMA,
        ],
    )(src, dst, gflat)


@jax.jit
def _run(src, dst, model):
    mp = jnp.zeros((NPAD, D_FEAT), jnp.bfloat16)
    mp = lax.dynamic_update_slice(mp, model.astype(jnp.bfloat16), (0, 0))
    g = _gram(mp)
    gflat = lax.bitcast_convert_type(g, jnp.int32).reshape(-1)
    out = _sc_gather(src, dst, gflat)
    return lax.bitcast_convert_type(out, jnp.float32)


def kernel(model, edge_index):
    ei = edge_index.astype(jnp.int32)
    return _run(ei[0], ei[1], model)


# speedup vs baseline: 1.5455x; 1.5455x over previous
import functools

import jax
import jax.numpy as jnp
from jax import lax
from jax.experimental import pallas as pl
from jax.experimental.pallas import tpu as pltpu
from jax.experimental.pallas import tpu_sc as plsc

N_NODES = 10000
N_EDGES = 320000
D_FEAT = 128
LANES = 16

NC = 2
NS = 16
NW = NC * NS

NPAD = 10240
BM = 512
BN = 2560
EDGES_PER_TILE = N_EDGES // NW


def _mm_body(m_ref, mt_ref, g_ref):
    g_ref[...] = lax.dot_general(
        m_ref[...], mt_ref[...], (((1,), (1,)), ((), ())),
        preferred_element_type=jnp.float32)


def _gram(mp):
    return pl.pallas_call(
        _mm_body,
        grid=(NPAD // BM, NPAD // BN),
        in_specs=[pl.BlockSpec((BM, D_FEAT), lambda i, j: (i, 0)),
                  pl.BlockSpec((BN, D_FEAT), lambda i, j: (j, 0))],
        out_specs=pl.BlockSpec((BM, BN), lambda i, j: (i, j)),
        out_shape=jax.ShapeDtypeStruct((NPAD, NPAD), jnp.float32),
    )(mp, mp)


def _sc_body(src_hbm, dst_hbm, g_hbm, out_hbm, sidx, didx, widx, vals, sem):
    cid = lax.axis_index("c")
    sid = lax.axis_index("s")
    wid = sid * NC + cid
    base = wid * EDGES_PER_TILE

    pltpu.sync_copy(src_hbm.at[pl.ds(base, EDGES_PER_TILE)], sidx)
    pltpu.sync_copy(dst_hbm.at[pl.ds(base, EDGES_PER_TILE)], didx)

    def flat_step(i, _):
        s = pl.ds(i * LANES, LANES)
        widx[s] = sidx[s] * NPAD + didx[s]
        return 0

    lax.fori_loop(0, EDGES_PER_TILE // LANES, flat_step, 0)

    pltpu.async_copy(g_hbm.at[widx], vals, sem).wait()
    pltpu.sync_copy(vals, out_hbm.at[pl.ds(base, EDGES_PER_TILE)])


def _sc_gather(src, dst, g):
    mesh = plsc.VectorSubcoreMesh(core_axis_name="c", subcore_axis_name="s")
    return pl.kernel(
        _sc_body,
        out_type=jax.ShapeDtypeStruct((N_EDGES,), jnp.float32),
        mesh=mesh,
        compiler_params=pltpu.CompilerParams(needs_layout_passes=False,
                                             use_tc_tiling_on_sc=False),
        scratch_types=[
            pltpu.VMEM((EDGES_PER_TILE,), jnp.int32),
            pltpu.VMEM((EDGES_PER_TILE,), jnp.int32),
            pltpu.VMEM((EDGES_PER_TILE,), jnp.int32),
            pltpu.VMEM((EDGES_PER_TILE,), jnp.float32),
            pltpu.SemaphoreType.DMA,
        ],
    )(src, dst, g)


@jax.jit
def _run(src, dst, model):
    mp = jnp.zeros((NPAD, D_FEAT), jnp.bfloat16)
    mp = lax.dynamic_update_slice(mp, model.astype(jnp.bfloat16), (0, 0))
    g = _gram(mp)
    gflat = g.reshape(NPAD * NPAD)
    return _sc_gather(src, dst, gflat)


def kernel(model, edge_index):
    ei = edge_index.astype(jnp.int32)
    return _run(ei[0], ei[1], model)
